# baseline (speedup 1.0000x reference)
import functools

import jax
import jax.numpy as jnp
from jax import lax
from jax.experimental import pallas as pl
from jax.experimental.pallas import tpu as pltpu
from jax.experimental.pallas import tpu_sc as plsc

_N = 10000
_E = 320000
_D = 128
_NW = 32
_EW = 10240
_EPAD = _NW * _EW
_NPAD = 10240
_RPT = _NPAD // 16
_BR = 1024


def _make_sc_aggregate(with_deg):
    mesh = plsc.VectorSubcoreMesh(core_axis_name="c", subcore_axis_name="s")
    _K = 32
    _NB = 4
    _C = _EW // _K
    part_t = jax.ShapeDtypeStruct((2, _NPAD, _D), jnp.float32)
    deg_t = jax.ShapeDtypeStruct((2, _NPAD // _D, _D), jnp.float32)
    out_type = (part_t, deg_t) if with_deg else part_t
    scratch = [
        pltpu.VMEM((_EW,), jnp.int32),
        pltpu.VMEM((_EW,), jnp.int32),
        pltpu.VMEM((_NB, _K, _D), jnp.float32),
    ] + [pltpu.SemaphoreType.DMA] * (2 * _NB) + [
        pltpu.VMEM_SHARED((_NPAD, _D), jnp.float32),
    ]
    if with_deg:
        scratch += [
            pltpu.VMEM((_NPAD,), jnp.float32),
            pltpu.VMEM_SHARED((_NPAD // _D, _D), jnp.float32),
            pltpu.VMEM((_NPAD // _D,), jnp.int32),
        ]

    @functools.partial(
        pl.kernel, out_type=out_type, mesh=mesh, scratch_types=scratch,
        compiler_params=pltpu.CompilerParams(needs_layout_passes=False))
    def agg(feat, srcp, dstp, *rest):
        if with_deg:
            part, degp = rest[0], rest[1]
            rest = rest[2:]
            src_v, dst_v, gbuf = rest[0], rest[1], rest[2]
            gs = rest[3:3 + _NB]
            ss = rest[3 + _NB:3 + 2 * _NB]
            acc, deg_v, degs, rowidx = rest[3 + 2 * _NB:]
        else:
            part = rest[0]
            rest = rest[1:]
            src_v, dst_v, gbuf = rest[0], rest[1], rest[2]
            gs = rest[3:3 + _NB]
            ss = rest[3 + _NB:3 + 2 * _NB]
            acc = rest[3 + 2 * _NB]
        c = lax.axis_index("c")
        s = lax.axis_index("s")
        wid = s * 2 + c

        pltpu.sync_copy(srcp.at[wid], src_v)
        pltpu.sync_copy(dstp.at[wid], dst_v)

        def zrow(i, carry):
            for j in range(_D // 16):
                gbuf[0, i, pl.ds(16 * j, 16)] = jnp.zeros((16,), jnp.float32)
            return carry

        lax.fori_loop(0, _K, zrow, 0)
        for t in range(_RPT // _K):
            pltpu.sync_copy(gbuf.at[0], acc.at[pl.ds(s * _RPT + t * _K, _K)])

        _DR = _NPAD // _D
        if with_deg:
            def zdeg(i, carry):
                deg_v[pl.ds(16 * i, 16)] = jnp.zeros((16,), jnp.float32)
                return carry

            lax.fori_loop(0, _NPAD // 16, zdeg, 0)
            for i in range(_DR // 16):
                rowidx[pl.ds(16 * i, 16)] = lax.iota(jnp.int32, 16) + 16 * i

            @pl.when(s < _DR // 8)
            def _zero_degs():
                pltpu.sync_copy(gbuf.at[0, pl.ds(0, 8)],
                                degs.at[pl.ds(s * 8, 8)])
        plsc.subcore_barrier()

        ones16 = jnp.ones((16,), jnp.float32)

        def hist(j):
            for v in range(_K // 16):
                idx = dst_v[pl.ds(j * _K + 16 * v, 16)]
                plsc.addupdate_scatter(deg_v, [idx], ones16)

        def sidx(j):
            return src_v.at[pl.ds(j * _K, _K)]

        def didx(j):
            return dst_v.at[pl.ds(j * _K, _K)]

        def drain_scatter(sem):
            pltpu.make_async_copy(feat.at[pl.ds(0, _K)],
                                  acc.at[pl.ds(0, _K)], sem).wait()

        pltpu.async_copy(feat.at[sidx(0)], gbuf.at[0], gs[0])
        pltpu.async_copy(feat.at[sidx(1)], gbuf.at[1], gs[1])

        def rnd(r, carry):
            for b in range(_NB):
                j = r * _NB + b
                bg = (b + 2) % _NB
                jg = j + 2

                @pl.when(jg < _C)
                def _issue_gather():
                    pltpu.async_copy(feat.at[sidx(jg)], gbuf.at[bg], gs[bg])

                if with_deg:
                    hist(j)
                pltpu.make_async_copy(feat.at[sidx(j)], gbuf.at[b],
                                      gs[b]).wait()
                pltpu.sync_copy(gbuf.at[b], acc.at[didx(j)], add=True)
            return carry

        lax.fori_loop(0, _C // _NB, rnd, 0)

        if with_deg:
            _HP = 16

            for h in range(_DR // _HP):
                gb = h % _NB

                def to2d(i, carry):
                    r = h * _HP + i
                    for j in range(_D // 16):
                        gbuf[gb, i, pl.ds(16 * j, 16)] = (
                            deg_v[pl.ds(_D * r + 16 * j, 16)])
                    return carry

                lax.fori_loop(0, _HP, to2d, 0)
                pltpu.sync_copy(gbuf.at[gb, pl.ds(0, _HP)],
                                degs.at[rowidx.at[pl.ds(h * _HP, _HP)]],
                                add=True)
        plsc.subcore_barrier()

        pltpu.sync_copy(acc.at[pl.ds(s * _RPT, _RPT)],
                        part.at[c, pl.ds(s * _RPT, _RPT)])
        if with_deg:
            @pl.when(s < _DR // 8)
            def _dump_degs():
                pltpu.sync_copy(degs.at[pl.ds(s * 8, 8)],
                                degp.at[c, pl.ds(s * 8, 8)])

    return agg


_sc_agg_deg = _make_sc_aggregate(True)
_sc_agg = _make_sc_aggregate(False)

_DN = (((1,), (1,)), ((), ()))


def _tc0_body(p0, p1, d0, d1, xb, wl, bl, wr, h_out, ideg_out):
    deg = d0[0] + d1[0]
    ideg = 1.0 / jnp.maximum(deg, 1.0)
    agg = (p0[0] + p1[0]) * ideg
    h = lax.dot_general(agg, wl[...], _DN, preferred_element_type=jnp.float32)
    h = h + bl[...] + lax.dot_general(xb[...], wr[...], _DN,
                                      preferred_element_type=jnp.float32)
    h_out[...] = jnp.maximum(h, 0.0)
    ideg_out[...] = ideg


def _tc_layer0(part, degp, xp, Wl0, bl0, Wr0):
    return pl.pallas_call(
        _tc0_body,
        grid=(_NPAD // _BR,),
        in_specs=[
            pl.BlockSpec((1, _BR, _D), lambda i: (0, i, 0)),
            pl.BlockSpec((1, _BR, _D), lambda i: (1, i, 0)),
            pl.BlockSpec((1, _BR, 1), lambda i: (0, i, 0)),
            pl.BlockSpec((1, _BR, 1), lambda i: (1, i, 0)),
            pl.BlockSpec((_BR, _D), lambda i: (i, 0)),
            pl.BlockSpec((_D, _D), lambda i: (0, 0)),
            pl.BlockSpec((1, _D), lambda i: (0, 0)),
            pl.BlockSpec((_D, _D), lambda i: (0, 0)),
        ],
        out_specs=[
            pl.BlockSpec((_BR, _D), lambda i: (i, 0)),
            pl.BlockSpec((_BR, 1), lambda i: (i, 0)),
        ],
        out_shape=[
            jax.ShapeDtypeStruct((_NPAD, _D), jnp.float32),
            jax.ShapeDtypeStruct((_NPAD, 1), jnp.float32),
        ],
    )(part, part, degp, degp, xp, Wl0, bl0.reshape(1, _D), Wr0)


def _tc1_body(p0, p1, hb, idb, wl, bl, wr, wo, bo, out):
    sblk = (p0[0] + p1[0]) * idb[...]
    h = lax.dot_general(sblk, wl[...], _DN, preferred_element_type=jnp.float32)
    h = h + bl[...] + lax.dot_general(hb[...], wr[...], _DN,
                                      preferred_element_type=jnp.float32)
    h = jnp.maximum(h, 0.0)
    out[...] = lax.dot_general(h, wo[...], _DN,
                               preferred_element_type=jnp.float32) + bo[...]


def _tc_layer1(part, h1, ideg, Wl1, bl1, Wr1, Wout, bout):
    return pl.pallas_call(
        _tc1_body,
        grid=(_NPAD // _BR,),
        in_specs=[
            pl.BlockSpec((1, _BR, _D), lambda i: (0, i, 0)),
            pl.BlockSpec((1, _BR, _D), lambda i: (1, i, 0)),
            pl.BlockSpec((_BR, _D), lambda i: (i, 0)),
            pl.BlockSpec((_BR, 1), lambda i: (i, 0)),
            pl.BlockSpec((_D, _D), lambda i: (0, 0)),
            pl.BlockSpec((1, _D), lambda i: (0, 0)),
            pl.BlockSpec((_D, _D), lambda i: (0, 0)),
            pl.BlockSpec((_D, _D), lambda i: (0, 0)),
            pl.BlockSpec((1, _D), lambda i: (0, 0)),
        ],
        out_specs=pl.BlockSpec((_BR, _D), lambda i: (i, 0)),
        out_shape=jax.ShapeDtypeStruct((_NPAD, _D), jnp.float32),
    )(part, part, h1, ideg, Wl1, bl1.reshape(1, _D), Wr1, Wout,
      bout.reshape(1, _D))


def kernel(x, edge_index, Wl0, bl0, Wr0, Wl1, bl1, Wr1, Wout, bout):
    src = edge_index[0].astype(jnp.int32)
    dst = edge_index[1].astype(jnp.int32)
    order = jnp.argsort(dst)
    src = src[order]
    dst = dst[order]
    pad = _EPAD - _E
    srcp = jnp.concatenate([src, jnp.zeros((pad,), jnp.int32)]).reshape(_NW, _EW)
    dstp = jnp.concatenate([dst, jnp.full((pad,), _N, jnp.int32)]).reshape(_NW, _EW)
    xp = jnp.concatenate(
        [x, jnp.zeros((_NPAD - _N, _D), jnp.float32)], axis=0)

    part1, degp = _sc_agg_deg(xp, srcp, dstp)
    h1, ideg = _tc_layer0(part1, degp.reshape(2, _NPAD, 1), xp, Wl0, bl0, Wr0)
    part2 = _sc_agg(h1, srcp, dstp)
    out = _tc_layer1(part2, h1, ideg, Wl1, bl1, Wr1, Wout, bout)
    return out[:_N]


# speedup vs baseline: 1.2330x; 1.2330x over previous
import functools

import jax
import jax.numpy as jnp
from jax import lax
from jax.experimental import pallas as pl
from jax.experimental.pallas import tpu as pltpu
from jax.experimental.pallas import tpu_sc as plsc

_N = 10000
_E = 320000
_D = 128
_NW = 32
_EW = 10240
_EPAD = _NW * _EW
_NPAD = 10240
_RPT = _NPAD // 16
_BR = 1024


def _make_sc_aggregate(with_deg):
    mesh = plsc.VectorSubcoreMesh(core_axis_name="c", subcore_axis_name="s")
    _K = 32
    _NB = 4
    _C = _EW // _K
    part_t = jax.ShapeDtypeStruct((2, _NPAD, _D), jnp.float32)
    deg_t = jax.ShapeDtypeStruct((2, _NPAD // _D, _D), jnp.float32)
    out_type = (part_t, deg_t) if with_deg else part_t
    scratch = [
        pltpu.VMEM((_EW,), jnp.int32),
        pltpu.VMEM((_EW,), jnp.int32),
        pltpu.VMEM((_NB, _K, _D), jnp.float32),
    ] + [pltpu.SemaphoreType.DMA] * _NB + [
        pltpu.VMEM_SHARED((_NPAD, _D), jnp.float32),
    ]
    if with_deg:
        scratch += [
            pltpu.VMEM((_NPAD,), jnp.float32),
            pltpu.VMEM_SHARED((_NPAD // _D, _D), jnp.float32),
            pltpu.VMEM((_NPAD // _D,), jnp.int32),
        ]

    @functools.partial(
        pl.kernel, out_type=out_type, mesh=mesh, scratch_types=scratch,
        compiler_params=pltpu.CompilerParams(needs_layout_passes=False))
    def agg(feat, srcp, dstp, *rest):
        if with_deg:
            part, degp = rest[0], rest[1]
            rest = rest[2:]
            src_v, dst_v, gbuf = rest[0], rest[1], rest[2]
            gs = rest[3:3 + _NB]
            acc, deg_v, degs, rowidx = rest[3 + _NB:]
        else:
            part = rest[0]
            rest = rest[1:]
            src_v, dst_v, gbuf = rest[0], rest[1], rest[2]
            gs = rest[3:3 + _NB]
            acc = rest[3 + _NB]
        c = lax.axis_index("c")
        s = lax.axis_index("s")
        wid = s * 2 + c

        pltpu.sync_copy(srcp.at[wid], src_v)
        pltpu.sync_copy(dstp.at[wid], dst_v)

        def zrow(i, carry):
            for j in range(_D // 16):
                gbuf[0, i, pl.ds(16 * j, 16)] = jnp.zeros((16,), jnp.float32)
            return carry

        lax.fori_loop(0, _K, zrow, 0)
        for t in range(_RPT // _K):
            pltpu.sync_copy(gbuf.at[0], acc.at[pl.ds(s * _RPT + t * _K, _K)])

        _DR = _NPAD // _D
        if with_deg:
            def zdeg(i, carry):
                deg_v[pl.ds(16 * i, 16)] = jnp.zeros((16,), jnp.float32)
                return carry

            lax.fori_loop(0, _NPAD // 16, zdeg, 0)
            for i in range(_DR // 16):
                rowidx[pl.ds(16 * i, 16)] = lax.iota(jnp.int32, 16) + 16 * i

            @pl.when(s < _DR // 8)
            def _zero_degs():
                pltpu.sync_copy(gbuf.at[0, pl.ds(0, 8)],
                                degs.at[pl.ds(s * 8, 8)])
        plsc.subcore_barrier()

        ones16 = jnp.ones((16,), jnp.float32)

        def hist(j):
            for v in range(_K // 16):
                idx = dst_v[pl.ds(j * _K + 16 * v, 16)]
                plsc.addupdate_scatter(deg_v, [idx], ones16)

        def sidx(j):
            return src_v.at[pl.ds(j * _K, _K)]

        def didx(j):
            return dst_v.at[pl.ds(j * _K, _K)]

        pltpu.async_copy(feat.at[sidx(0)], gbuf.at[0], gs[0])
        pltpu.async_copy(feat.at[sidx(1)], gbuf.at[1], gs[1])

        def rnd(r, carry):
            for b in range(_NB):
                j = r * _NB + b
                bg = (b + 2) % _NB
                jg = j + 2

                @pl.when(jg < _C)
                def _issue_gather():
                    pltpu.async_copy(feat.at[sidx(jg)], gbuf.at[bg], gs[bg])

                if with_deg:
                    hist(j)
                pltpu.make_async_copy(feat.at[sidx(j)], gbuf.at[b],
                                      gs[b]).wait()
                pltpu.sync_copy(gbuf.at[b], acc.at[didx(j)], add=True)
            return carry

        lax.fori_loop(0, _C // _NB, rnd, 0)

        if with_deg:
            _HP = 16

            for h in range(_DR // _HP):
                gb = h % _NB

                def to2d(i, carry):
                    r = h * _HP + i
                    for j in range(_D // 16):
                        gbuf[gb, i, pl.ds(16 * j, 16)] = (
                            deg_v[pl.ds(_D * r + 16 * j, 16)])
                    return carry

                lax.fori_loop(0, _HP, to2d, 0)
                pltpu.sync_copy(gbuf.at[gb, pl.ds(0, _HP)],
                                degs.at[rowidx.at[pl.ds(h * _HP, _HP)]],
                                add=True)
        plsc.subcore_barrier()

        pltpu.sync_copy(acc.at[pl.ds(s * _RPT, _RPT)],
                        part.at[c, pl.ds(s * _RPT, _RPT)])
        if with_deg:
            @pl.when(s < _DR // 8)
            def _dump_degs():
                pltpu.sync_copy(degs.at[pl.ds(s * 8, 8)],
                                degp.at[c, pl.ds(s * 8, 8)])

    return agg


_sc_agg_deg = _make_sc_aggregate(True)
_sc_agg = _make_sc_aggregate(False)

_DN = (((1,), (1,)), ((), ()))


def _tc0_body(p0, p1, d0, d1, xb, wl, bl, wr, h_out, ideg_out):
    deg = d0[0] + d1[0]
    ideg = 1.0 / jnp.maximum(deg, 1.0)
    agg = (p0[0] + p1[0]) * ideg
    h = lax.dot_general(agg, wl[...], _DN, preferred_element_type=jnp.float32)
    h = h + bl[...] + lax.dot_general(xb[...], wr[...], _DN,
                                      preferred_element_type=jnp.float32)
    h_out[...] = jnp.maximum(h, 0.0)
    ideg_out[...] = ideg


def _tc_layer0(part, degp, xp, Wl0, bl0, Wr0):
    return pl.pallas_call(
        _tc0_body,
        grid=(_NPAD // _BR,),
        in_specs=[
            pl.BlockSpec((1, _BR, _D), lambda i: (0, i, 0)),
            pl.BlockSpec((1, _BR, _D), lambda i: (1, i, 0)),
            pl.BlockSpec((1, _BR, 1), lambda i: (0, i, 0)),
            pl.BlockSpec((1, _BR, 1), lambda i: (1, i, 0)),
            pl.BlockSpec((_BR, _D), lambda i: (i, 0)),
            pl.BlockSpec((_D, _D), lambda i: (0, 0)),
            pl.BlockSpec((1, _D), lambda i: (0, 0)),
            pl.BlockSpec((_D, _D), lambda i: (0, 0)),
        ],
        out_specs=[
            pl.BlockSpec((_BR, _D), lambda i: (i, 0)),
            pl.BlockSpec((_BR, 1), lambda i: (i, 0)),
        ],
        out_shape=[
            jax.ShapeDtypeStruct((_NPAD, _D), jnp.float32),
            jax.ShapeDtypeStruct((_NPAD, 1), jnp.float32),
        ],
    )(part, part, degp, degp, xp, Wl0, bl0.reshape(1, _D), Wr0)


def _tc1_body(p0, p1, hb, idb, wl, bl, wr, wo, bo, out):
    sblk = (p0[0] + p1[0]) * idb[...]
    h = lax.dot_general(sblk, wl[...], _DN, preferred_element_type=jnp.float32)
    h = h + bl[...] + lax.dot_general(hb[...], wr[...], _DN,
                                      preferred_element_type=jnp.float32)
    h = jnp.maximum(h, 0.0)
    out[...] = lax.dot_general(h, wo[...], _DN,
                               preferred_element_type=jnp.float32) + bo[...]


def _tc_layer1(part, h1, ideg, Wl1, bl1, Wr1, Wout, bout):
    return pl.pallas_call(
        _tc1_body,
        grid=(_NPAD // _BR,),
        in_specs=[
            pl.BlockSpec((1, _BR, _D), lambda i: (0, i, 0)),
            pl.BlockSpec((1, _BR, _D), lambda i: (1, i, 0)),
            pl.BlockSpec((_BR, _D), lambda i: (i, 0)),
            pl.BlockSpec((_BR, 1), lambda i: (i, 0)),
            pl.BlockSpec((_D, _D), lambda i: (0, 0)),
            pl.BlockSpec((1, _D), lambda i: (0, 0)),
            pl.BlockSpec((_D, _D), lambda i: (0, 0)),
            pl.BlockSpec((_D, _D), lambda i: (0, 0)),
            pl.BlockSpec((1, _D), lambda i: (0, 0)),
        ],
        out_specs=pl.BlockSpec((_BR, _D), lambda i: (i, 0)),
        out_shape=jax.ShapeDtypeStruct((_NPAD, _D), jnp.float32),
    )(part, part, h1, ideg, Wl1, bl1.reshape(1, _D), Wr1, Wout,
      bout.reshape(1, _D))


def kernel(x, edge_index, Wl0, bl0, Wr0, Wl1, bl1, Wr1, Wout, bout):
    src = edge_index[0].astype(jnp.int32)
    dst = edge_index[1].astype(jnp.int32)
    pad = _EPAD - _E
    srcp = jnp.concatenate([src, jnp.zeros((pad,), jnp.int32)]).reshape(_NW, _EW)
    dstp = jnp.concatenate([dst, jnp.full((pad,), _N, jnp.int32)]).reshape(_NW, _EW)
    xp = jnp.concatenate(
        [x, jnp.zeros((_NPAD - _N, _D), jnp.float32)], axis=0)

    part1, degp = _sc_agg_deg(xp, srcp, dstp)
    h1, ideg = _tc_layer0(part1, degp.reshape(2, _NPAD, 1), xp, Wl0, bl0, Wr0)
    part2 = _sc_agg(h1, srcp, dstp)
    out = _tc_layer1(part2, h1, ideg, Wl1, bl1, Wr1, Wout, bout)
    return out[:_N]
